# baseline (speedup 1.0000x reference)
import jax
import jax.numpy as jnp
from jax import lax
from jax.experimental import pallas as pl
from jax.experimental.pallas import tpu as pltpu
from jax.experimental.pallas import tpu_sc as plsc

NC = 2
NS = 16
NW = NC * NS
L = 16

B = 16384
POS = 20
NEG = 200
K = POS + NEG
KP = 224
E = 64
CB = 2
NG = 4
GW = (CB * K) // NG
CR = CB * K
BW = B // NW
CHUNKS = BW // CB
CIDX_GW = 128
CIDX_NG = BW // CIDX_GW


def _sc_body(labels_hbm, cidx_hbm, inemb_hbm, outemb_hbm, dots_hbm,
             cidx_v, centers_v, idx_a, idx_b, rows_a, rows_b,
             dots_a, dots_b, stage_v, sem_rows_a, sem_rows_b, sem_idx,
             sem_dots_a, sem_dots_b):
    wid = lax.axis_index("s") * NC + lax.axis_index("c")
    chunk0 = wid * CHUNKS

    pltpu.sync_copy(cidx_hbm.at[wid], cidx_v)
    for g in range(CIDX_NG):
        pltpu.async_copy(inemb_hbm.at[cidx_v.at[g]], centers_v.at[g],
                         sem_rows_a).wait()

    idx_bufs = (idx_a, idx_b)
    rows_bufs = (rows_a, rows_b)
    dots_bufs = (dots_a, dots_b)
    sem_rows = (sem_rows_a, sem_rows_b)
    sem_dots = (sem_dots_a, sem_dots_b)
    lanes = lax.broadcasted_iota(jnp.int32, (L,), 0)

    def fire_gathers(idx_v, rows_v, sem):
        for g in range(NG):
            pltpu.async_copy(outemb_hbm.at[idx_v.at[g]],
                             rows_v.at[pl.ds(g * GW, GW)], sem)

    def drain_gathers(idx_v, rows_v, sem):
        for g in range(NG):
            pltpu.make_async_copy(outemb_hbm.at[idx_v.at[g]],
                                  rows_v.at[pl.ds(g * GW, GW)],
                                  sem).wait()

    pltpu.sync_copy(labels_hbm.at[chunk0], idx_a)
    fire_gathers(idx_a, rows_a, sem_rows_a)
    pltpu.async_copy(labels_hbm.at[chunk0 + 1], idx_b, sem_idx)

    def outer_body(c2, carry):
        for sub in range(2):
            c = c2 * 2 + sub
            buf = sub
            nbuf = 1 - sub
            idx_c, rows_c, dots_c = idx_bufs[buf], rows_bufs[buf], dots_bufs[buf]
            idx_n, rows_n = idx_bufs[nbuf], rows_bufs[nbuf]

            @pl.when(c + 1 < CHUNKS)
            def _():
                pltpu.make_async_copy(labels_hbm.at[chunk0 + c + 1], idx_n,
                                      sem_idx).wait()
                fire_gathers(idx_n, rows_n, sem_rows[nbuf])

            drain_gathers(idx_c, rows_c, sem_rows[buf])

            @pl.when(c + 2 < CHUNKS)
            def _():
                pltpu.async_copy(labels_hbm.at[chunk0 + c + 2], idx_c, sem_idx)

            @pl.when(c >= 2)
            def _():
                pltpu.make_async_copy(dots_c, dots_hbm.at[chunk0],
                                      sem_dots[buf]).wait()

            for b_local in range(CB):
                cb = c * CB + b_local
                chi = cb // CIDX_GW
                clo = cb % CIDX_GW
                cvec = [centers_v[chi, clo, pl.ds(s * L, L)]
                        for s in range(E // L)]

                def grp_body(jg, _, b_local=b_local, rows_c=rows_c,
                             dots_c=dots_c, cvec=cvec):
                    for u in range(L):
                        r = jnp.minimum(jg * L + u, K - 1)
                        base = b_local * K + r
                        segs = [rows_c[base, pl.ds(s * L, L)]
                                for s in range(E // L)]
                        p = ((segs[0] * cvec[0] + segs[1] * cvec[1])
                             + (segs[2] * cvec[2] + segs[3] * cvec[3]))
                        stage_v[u, pl.ds(0, L)] = p
                    acc0 = jnp.zeros((L,), jnp.float32)
                    acc1 = jnp.zeros((L,), jnp.float32)
                    for col in range(L):
                        g = plsc.load_gather(
                            stage_v, [lanes, jnp.full((L,), col, jnp.int32)])
                        if col % 2 == 0:
                            acc0 = acc0 + g
                        else:
                            acc1 = acc1 + g
                    dots_c[b_local, pl.ds(jg * L, L)] = acc0 + acc1
                    return 0

                lax.fori_loop(0, KP // L, grp_body, 0)

            pltpu.async_copy(dots_c, dots_hbm.at[chunk0 + c], sem_dots[buf])
        return carry

    lax.fori_loop(0, CHUNKS // 2, outer_body, 0)

    for buf in range(2):
        pltpu.make_async_copy(dots_bufs[buf], dots_hbm.at[chunk0],
                              sem_dots[buf]).wait()


@jax.jit
def _sc_dots(labels3, cidx, in_embed, out_embed):
    mesh = plsc.VectorSubcoreMesh(core_axis_name="c", subcore_axis_name="s")
    return pl.kernel(
        _sc_body,
        out_type=jax.ShapeDtypeStruct((B // CB, CB, KP), jnp.float32),
        mesh=mesh,
        scratch_types=[
            pltpu.VMEM((CIDX_NG, CIDX_GW), jnp.int32),
            pltpu.VMEM((CIDX_NG, CIDX_GW, E), jnp.float32),
            pltpu.VMEM((NG, GW), jnp.int32),
            pltpu.VMEM((NG, GW), jnp.int32),
            pltpu.VMEM((CR, E), jnp.float32),
            pltpu.VMEM((CR, E), jnp.float32),
            pltpu.VMEM((CB, KP), jnp.float32),
            pltpu.VMEM((CB, KP), jnp.float32),
            pltpu.VMEM((L, L + 1), jnp.float32),
            pltpu.SemaphoreType.DMA,
            pltpu.SemaphoreType.DMA,
            pltpu.SemaphoreType.DMA,
            pltpu.SemaphoreType.DMA,
            pltpu.SemaphoreType.DMA,
        ],
        compiler_params=pltpu.CompilerParams(
            use_tc_tiling_on_sc=False, needs_layout_passes=False),
    )(labels3, cidx, in_embed, out_embed)


def _tc_loss_body(dots_ref, out_ref):
    d = dots_ref[...]
    colidx = lax.broadcasted_iota(jnp.int32, d.shape, 1)
    x = jnp.where(colidx < POS, d, -d)
    ls = jax.nn.log_sigmoid(x)
    ls = jnp.where(colidx < K, ls, 0.0)
    out_ref[...] = -jnp.sum(ls, axis=1)


@jax.jit
def _tc_loss(dots2):
    blk = 2048
    return pl.pallas_call(
        _tc_loss_body,
        grid=(B // blk,),
        in_specs=[pl.BlockSpec((blk, KP), lambda i: (i, 0))],
        out_specs=pl.BlockSpec((blk,), lambda i: (i,)),
        out_shape=jax.ShapeDtypeStruct((B,), jnp.float32),
    )(dots2)


def kernel(input_labels, pos_labels, neg_labels, in_embed, out_embed):
    labels = jnp.concatenate(
        [pos_labels.astype(jnp.int32), neg_labels.astype(jnp.int32)], axis=1)
    labels3 = labels.reshape(B // CB, NG, GW)
    cidx = input_labels.astype(jnp.int32).reshape(NW, CIDX_NG, CIDX_GW)
    dots = _sc_dots(labels3, cidx, in_embed, out_embed)
    return _tc_loss(dots.reshape(B, KP))


# speedup vs baseline: 1.0192x; 1.0192x over previous
import jax
import jax.numpy as jnp
from jax import lax
from jax.experimental import pallas as pl
from jax.experimental.pallas import tpu as pltpu
from jax.experimental.pallas import tpu_sc as plsc

NC = 2
NS = 16
NW = NC * NS
L = 16

B = 16384
POS = 20
NEG = 200
K = POS + NEG
KP = 224
E = 64
CB = 2
NGH = 2
GH = NEG // NGH
CR = CB * K
BW = B // NW
CHUNKS = BW // CB
CIDX_GW = 128
CIDX_NG = BW // CIDX_GW


def _sc_body(cidx_hbm, pos_hbm, neg_hbm, inemb_hbm, outemb_hbm, loss_hbm,
             cidx_v, centers_v, idxp_a, idxp_b, idxn_a, idxn_b,
             rows_a, rows_b, stage_v, loss_v,
             sem_rows_a, sem_rows_b, sem_idx):
    wid = lax.axis_index("s") * NC + lax.axis_index("c")
    b0w = wid * BW

    pltpu.sync_copy(cidx_hbm.at[wid], cidx_v)
    for g in range(CIDX_NG):
        pltpu.async_copy(inemb_hbm.at[cidx_v.at[g]], centers_v.at[g],
                         sem_rows_a).wait()

    idxp_bufs = (idxp_a, idxp_b)
    idxn_bufs = (idxn_a, idxn_b)
    rows_bufs = (rows_a, rows_b)
    sem_rows = (sem_rows_a, sem_rows_b)
    lanes = lax.broadcasted_iota(jnp.int32, (L,), 0)
    mask15 = lanes == (L - 1)

    def idx_copies(b0, idxp_v, idxn_v, start):
        op = pltpu.async_copy if start else pltpu.make_async_copy
        return (op(pos_hbm.at[pl.ds(b0, CB)], idxp_v, sem_idx),
                op(neg_hbm.at[pl.ds(b0, CB)], idxn_v, sem_idx))

    def gathers(idxp_v, idxn_v, rows_v, sem, start):
        op = pltpu.async_copy if start else pltpu.make_async_copy
        cps = []
        for bl in range(CB):
            base = bl * K
            cps.append(op(outemb_hbm.at[idxp_v.at[bl]],
                          rows_v.at[pl.ds(base, POS)], sem))
            for g in range(NGH):
                cps.append(op(outemb_hbm.at[idxn_v.at[bl, g]],
                              rows_v.at[pl.ds(base + POS + g * GH, GH)], sem))
        return cps

    pltpu.sync_copy(pos_hbm.at[pl.ds(b0w, CB)], idxp_a)
    pltpu.sync_copy(neg_hbm.at[pl.ds(b0w, CB)], idxn_a)
    gathers(idxp_a, idxn_a, rows_a, sem_rows_a, start=True)
    idx_copies(b0w + CB, idxp_b, idxn_b, start=True)

    def outer_body(c2, carry):
        for sub in range(2):
            c = c2 * 2 + sub
            buf = sub
            nbuf = 1 - sub
            idxp_c, idxn_c, rows_c = idxp_bufs[buf], idxn_bufs[buf], rows_bufs[buf]
            idxp_n, idxn_n, rows_n = idxp_bufs[nbuf], idxn_bufs[nbuf], rows_bufs[nbuf]
            b0 = b0w + c * CB

            @pl.when(c + 1 < CHUNKS)
            def _():
                for cp in idx_copies(b0 + CB, idxp_n, idxn_n, start=False):
                    cp.wait()
                gathers(idxp_n, idxn_n, rows_n, sem_rows[nbuf], start=True)

            for cp in gathers(idxp_c, idxn_c, rows_c, sem_rows[buf],
                              start=False):
                cp.wait()

            @pl.when(c + 2 < CHUNKS)
            def _():
                idx_copies(b0 + 2 * CB, idxp_c, idxn_c, start=True)

            for b_local in range(CB):
                cb = c * CB + b_local
                chi = cb // CIDX_GW
                clo = cb % CIDX_GW
                cvec = [centers_v[chi, clo, pl.ds(s * L, L)]
                        for s in range(E // L)]

                def grp_body(jg, lacc, b_local=b_local, rows_c=rows_c,
                             cvec=cvec):
                    for u in range(L):
                        r = jnp.minimum(jg * L + u, K - 1)
                        base = b_local * K + r
                        segs = [rows_c[base, pl.ds(s * L, L)]
                                for s in range(E // L)]
                        p = ((segs[0] * cvec[0] + segs[1] * cvec[1])
                             + (segs[2] * cvec[2] + segs[3] * cvec[3]))
                        stage_v[u, pl.ds(0, L)] = p
                    acc0 = jnp.zeros((L,), jnp.float32)
                    acc1 = jnp.zeros((L,), jnp.float32)
                    for col in range(L):
                        gth = plsc.load_gather(
                            stage_v, [lanes, jnp.full((L,), col, jnp.int32)])
                        if col % 2 == 0:
                            acc0 = acc0 + gth
                        else:
                            acc1 = acc1 + gth
                    dot = acc0 + acc1
                    jvec = jnp.broadcast_to(jg * L, (L,)).astype(jnp.int32) + lanes
                    x = jnp.where(jvec < POS, dot, -dot)
                    ea = jnp.exp(-jnp.abs(x))
                    s = ea / (2.0 + ea)
                    s2 = s * s
                    poly = 1.0 + s2 * ((1.0 / 3.0) + s2 * ((1.0 / 5.0)
                           + s2 * ((1.0 / 7.0) + s2 * (1.0 / 9.0))))
                    ls = jnp.minimum(x, 0.0) - 2.0 * s * poly
                    ls = jnp.where(jvec < K, ls, 0.0)
                    return lacc + ls

                lacc = lax.fori_loop(0, KP // L, grp_body,
                                     jnp.zeros((L,), jnp.float32))
                cs = plsc.cumsum(-lacc)
                plsc.store_scatter(loss_v, [jnp.full((L,), cb, jnp.int32)],
                                   cs, mask=mask15)
        return carry

    lax.fori_loop(0, CHUNKS // 2, outer_body, 0)

    pltpu.sync_copy(loss_v, loss_hbm.at[wid])


@jax.jit
def _sc_loss(cidx, pos3, neg3, in_embed, out_embed):
    mesh = plsc.VectorSubcoreMesh(core_axis_name="c", subcore_axis_name="s")
    return pl.kernel(
        _sc_body,
        out_type=jax.ShapeDtypeStruct((NW, BW), jnp.float32),
        mesh=mesh,
        scratch_types=[
            pltpu.VMEM((CIDX_NG, CIDX_GW), jnp.int32),
            pltpu.VMEM((CIDX_NG, CIDX_GW, E), jnp.float32),
            pltpu.VMEM((CB, POS), jnp.int32),
            pltpu.VMEM((CB, POS), jnp.int32),
            pltpu.VMEM((CB, NGH, GH), jnp.int32),
            pltpu.VMEM((CB, NGH, GH), jnp.int32),
            pltpu.VMEM((CR, E), jnp.float32),
            pltpu.VMEM((CR, E), jnp.float32),
            pltpu.VMEM((L, L + 1), jnp.float32),
            pltpu.VMEM((BW,), jnp.float32),
            pltpu.SemaphoreType.DMA,
            pltpu.SemaphoreType.DMA,
            pltpu.SemaphoreType.DMA,
        ],
        compiler_params=pltpu.CompilerParams(
            use_tc_tiling_on_sc=False, needs_layout_passes=False),
    )(cidx, pos3, neg3, in_embed, out_embed)


def kernel(input_labels, pos_labels, neg_labels, in_embed, out_embed):
    cidx = input_labels.astype(jnp.int32).reshape(NW, CIDX_NG, CIDX_GW)
    pos3 = pos_labels.astype(jnp.int32)
    neg3 = neg_labels.astype(jnp.int32).reshape(B, NGH, GH)
    loss = _sc_loss(cidx, pos3, neg3, in_embed, out_embed)
    return loss.reshape(B)


# speedup vs baseline: 1.0343x; 1.0148x over previous
import jax
import jax.numpy as jnp
from jax import lax
from jax.experimental import pallas as pl
from jax.experimental.pallas import tpu as pltpu
from jax.experimental.pallas import tpu_sc as plsc

NC = 2
NS = 16
NW = NC * NS
L = 16

B = 16384
POS = 20
NEG = 200
K = POS + NEG
KP = 224
E = 64
CB = 2
NGH = 2
GH = NEG // NGH
CR = CB * K
BW = B // NW
CHUNKS = BW // CB
CIDX_GW = 128
CIDX_NG = BW // CIDX_GW
NGRP_R = 7


def _sc_body(cidx_hbm, pos_hbm, neg_hbm, inemb_hbm, outemb_hbm, loss_hbm,
             cidx_v, centers_v, idxp_a, idxp_b, idxn_a, idxn_b,
             rows_a, rows_b, stage_v, loss_v,
             sem_rows_a, sem_rows_b, sem_idx):
    wid = lax.axis_index("s") * NC + lax.axis_index("c")
    b0w = wid * BW

    pltpu.sync_copy(cidx_hbm.at[wid], cidx_v)
    for g in range(CIDX_NG):
        pltpu.async_copy(inemb_hbm.at[cidx_v.at[g]], centers_v.at[g],
                         sem_rows_a).wait()

    idxp_bufs = (idxp_a, idxp_b)
    idxn_bufs = (idxn_a, idxn_b)
    rows_bufs = (rows_a, rows_b)
    sem_rows = (sem_rows_a, sem_rows_b)
    lanes = lax.broadcasted_iota(jnp.int32, (L,), 0)
    mask15 = lanes == (L - 1)

    def idx_copies(b0, idxp_v, idxn_v, start):
        op = pltpu.async_copy if start else pltpu.make_async_copy
        return (op(pos_hbm.at[pl.ds(b0, CB)], idxp_v, sem_idx),
                op(neg_hbm.at[pl.ds(b0, CB)], idxn_v, sem_idx))

    def gathers(idxp_v, idxn_v, rows_v, sem, start):
        op = pltpu.async_copy if start else pltpu.make_async_copy
        cps = []
        for bl in range(CB):
            base = bl * K
            cps.append(op(outemb_hbm.at[idxp_v.at[bl]],
                          rows_v.at[pl.ds(base, POS)], sem))
            for g in range(NGH):
                cps.append(op(outemb_hbm.at[idxn_v.at[bl, g]],
                              rows_v.at[pl.ds(base + POS + g * GH, GH)], sem))
        return cps

    pltpu.sync_copy(pos_hbm.at[pl.ds(b0w, CB)], idxp_a)
    pltpu.sync_copy(neg_hbm.at[pl.ds(b0w, CB)], idxn_a)
    gathers(idxp_a, idxn_a, rows_a, sem_rows_a, start=True)
    idx_copies(b0w + CB, idxp_b, idxn_b, start=True)

    def outer_body(c2, carry):
        for sub in range(2):
            c = c2 * 2 + sub
            buf = sub
            nbuf = 1 - sub
            idxp_c, idxn_c, rows_c = idxp_bufs[buf], idxn_bufs[buf], rows_bufs[buf]
            idxp_n, idxn_n, rows_n = idxp_bufs[nbuf], idxn_bufs[nbuf], rows_bufs[nbuf]
            b0 = b0w + c * CB

            @pl.when(c + 1 < CHUNKS)
            def _():
                for cp in idx_copies(b0 + CB, idxp_n, idxn_n, start=False):
                    cp.wait()
                gathers(idxp_n, idxn_n, rows_n, sem_rows[nbuf], start=True)

            for cp in gathers(idxp_c, idxn_c, rows_c, sem_rows[buf],
                              start=False):
                cp.wait()

            @pl.when(c + 2 < CHUNKS)
            def _():
                idx_copies(b0 + 2 * CB, idxp_c, idxn_c, start=True)

            for b_local in range(CB):
                cb = c * CB + b_local
                chi = cb // CIDX_GW
                clo = cb % CIDX_GW
                fchi = jnp.full((L,), 0, jnp.int32) + jnp.broadcast_to(
                    chi, (L,)).astype(jnp.int32)
                fclo = jnp.broadcast_to(clo, (L,)).astype(jnp.int32)
                for cc in range(E):
                    rot = (lanes + cc) & (E - 1)
                    v = plsc.load_gather(centers_v, [fchi, fclo, rot])
                    stage_v[cc, pl.ds(0, L)] = v

                def rnd_body(rnd, lacc, b_local=b_local, rows_c=rows_c):
                    bK = b_local * K
                    base0 = bK + rnd * (NGRP_R * L)
                    i0s = [jnp.minimum(base0 + g * L + lanes, bK + K - 1)
                           for g in range(NGRP_R)]
                    accs = [jnp.zeros((L,), jnp.float32)
                            for _ in range(NGRP_R)]
                    for cc in range(E):
                        cvr = stage_v[cc, pl.ds(0, L)]
                        rot = (lanes + cc) & (E - 1)
                        for g in range(NGRP_R):
                            gth = plsc.load_gather(rows_c, [i0s[g], rot])
                            accs[g] = accs[g] + gth * cvr
                    for g in range(NGRP_R):
                        jvec = jnp.broadcast_to(
                            rnd * (NGRP_R * L) + g * L, (L,)
                        ).astype(jnp.int32) + lanes
                        x = jnp.where(jvec < POS, accs[g], -accs[g])
                        ea = jnp.exp(-jnp.abs(x))
                        s = ea / (2.0 + ea)
                        s2 = s * s
                        poly = 1.0 + s2 * ((1.0 / 3.0) + s2 * ((1.0 / 5.0)
                               + s2 * ((1.0 / 7.0) + s2 * (1.0 / 9.0))))
                        ls = jnp.minimum(x, 0.0) - 2.0 * s * poly
                        lacc = lacc + jnp.where(jvec < K, ls, 0.0)
                    return lacc

                lacc = lax.fori_loop(0, KP // (NGRP_R * L), rnd_body,
                                     jnp.zeros((L,), jnp.float32))
                cs = plsc.cumsum(-lacc)
                plsc.store_scatter(loss_v, [jnp.full((L,), cb, jnp.int32)],
                                   cs, mask=mask15)
        return carry

    lax.fori_loop(0, CHUNKS // 2, outer_body, 0)

    pltpu.sync_copy(loss_v, loss_hbm.at[wid])


@jax.jit
def _sc_loss(cidx, pos3, neg3, in_embed, out_embed):
    mesh = plsc.VectorSubcoreMesh(core_axis_name="c", subcore_axis_name="s")
    return pl.kernel(
        _sc_body,
        out_type=jax.ShapeDtypeStruct((NW, BW), jnp.float32),
        mesh=mesh,
        scratch_types=[
            pltpu.VMEM((CIDX_NG, CIDX_GW), jnp.int32),
            pltpu.VMEM((CIDX_NG, CIDX_GW, E), jnp.float32),
            pltpu.VMEM((CB, POS), jnp.int32),
            pltpu.VMEM((CB, POS), jnp.int32),
            pltpu.VMEM((CB, NGH, GH), jnp.int32),
            pltpu.VMEM((CB, NGH, GH), jnp.int32),
            pltpu.VMEM((CR, E), jnp.float32),
            pltpu.VMEM((CR, E), jnp.float32),
            pltpu.VMEM((E, L), jnp.float32),
            pltpu.VMEM((BW,), jnp.float32),
            pltpu.SemaphoreType.DMA,
            pltpu.SemaphoreType.DMA,
            pltpu.SemaphoreType.DMA,
        ],
        compiler_params=pltpu.CompilerParams(
            use_tc_tiling_on_sc=False, needs_layout_passes=False),
    )(cidx, pos3, neg3, in_embed, out_embed)


def kernel(input_labels, pos_labels, neg_labels, in_embed, out_embed):
    cidx = input_labels.astype(jnp.int32).reshape(NW, CIDX_NG, CIDX_GW)
    pos3 = pos_labels.astype(jnp.int32)
    neg3 = neg_labels.astype(jnp.int32).reshape(B, NGH, GH)
    loss = _sc_loss(cidx, pos3, neg3, in_embed, out_embed)
    return loss.reshape(B)
